# baseline (speedup 1.0000x reference)
import functools

import jax
import jax.numpy as jnp
from jax import lax
from jax.experimental import pallas as pl
from jax.experimental.pallas import tpu as pltpu
from jax.experimental.pallas import tpu_sc as plsc

N, D, H = 4096, 1024, 128
EMA = 0.9
LANE = 16
NC, NS = 2, 16
NW = NC * NS

S_SC = 1024
ROWS_PER_W = S_SC // NW
N_TC = N - S_SC

BLKT = 1024
NFIX = N // LANE
NFIX_SC = S_SC // LANE


def _blend_body(x_ref, mask_ref, b1_ref, w2_ref, b2_ref, w3_ref, b3_ref,
                o_ref, flags_ref):
    h1 = jax.nn.relu(b1_ref[...])
    h2 = jax.nn.relu(
        lax.dot_general(h1, w2_ref[...], (((1,), (1,)), ((), ())),
                        precision=lax.Precision.HIGHEST) + b2_ref[...])
    pred = lax.dot_general(h2, w3_ref[...], (((1,), (1,)), ((), ())),
                           precision=lax.Precision.HIGHEST) + b3_ref[...]
    p01 = (1.0 - EMA) * pred
    o_ref[...] = EMA * x_ref[...] + p01
    m = mask_ref[...].astype(jnp.int32)
    complete = jnp.min(m, axis=1, keepdims=True)
    flags_ref[...] = complete.astype(jnp.float32).reshape(BLKT // LANE, LANE)


def _blend(X, mask_i8, b1, W2, b2, W3, b3):
    return pl.pallas_call(
        _blend_body,
        grid=(N // BLKT,),
        in_specs=[
            pl.BlockSpec((BLKT, D), lambda i: (i, 0)),
            pl.BlockSpec((BLKT, D), lambda i: (i, 0)),
            pl.BlockSpec((1, H), lambda i: (0, 0)),
            pl.BlockSpec((H, H), lambda i: (0, 0)),
            pl.BlockSpec((1, H), lambda i: (0, 0)),
            pl.BlockSpec((D, H), lambda i: (0, 0)),
            pl.BlockSpec((1, D), lambda i: (0, 0)),
        ],
        out_specs=[
            pl.BlockSpec((BLKT, D), lambda i: (i, 0)),
            pl.BlockSpec((BLKT // LANE, LANE), lambda i: (i, 0)),
        ],
        out_shape=[
            jax.ShapeDtypeStruct((N, D), jnp.float32),
            jax.ShapeDtypeStruct((N // LANE, LANE), jnp.float32),
        ],
    )(X, mask_i8, b1.reshape(1, H), W2, b2.reshape(1, H), W3,
      b3.reshape(1, D))


def _sc_flags_body(mask_hbm_2d, flags_hbm_2d, m_v, f_v, in_sem):
    wid = lax.axis_index("s") * NC + lax.axis_index("c")
    base = wid * ROWS_PER_W
    mask_hbm = mask_hbm_2d.reshape(N // 8, 8, D)
    flags_hbm = flags_hbm_2d.reshape(S_SC // 8, 8, 64)

    pltpu.async_copy(
        mask_hbm.at[pl.ds(base // 8, ROWS_PER_W // 8)], m_v, in_sem).wait()

    for q in range(ROWS_PER_W // 8):
        for k in range(8):
            vs = [m_v[q, k, pl.ds(w * 64, 64)] for w in range(LANE)]
            while len(vs) > 1:
                vs = [jnp.minimum(vs[i], vs[i + 1])
                      for i in range(0, len(vs), 2)]
            f_v[q, k] = vs[0]
    pltpu.sync_copy(f_v, flags_hbm.at[pl.ds(base // 8, ROWS_PER_W // 8)])


@functools.lru_cache(maxsize=1)
def _sc_flags():
    return pl.kernel(
        _sc_flags_body,
        out_type=jax.ShapeDtypeStruct((S_SC, 64), jnp.int8),
        mesh=plsc.VectorSubcoreMesh(core_axis_name="c", subcore_axis_name="s"),
        scratch_types=[
            pltpu.VMEM((ROWS_PER_W // 8, 8, D), jnp.int8),
            pltpu.VMEM((ROWS_PER_W // 8, 8, 64), jnp.int8),
            pltpu.SemaphoreType.DMA,
        ],
    )


def _fixup_body(blend_ref, x_ref, mf_ref, ftc_ref, o_ref, fl_s, sem):
    mf = mf_ref[...].astype(jnp.int32)
    rowmin = jnp.min(mf, axis=1)
    fl_s[0:NFIX_SC] = rowmin.reshape(NFIX_SC, LANE).astype(jnp.float32)
    fl_s[NFIX_SC:NFIX] = ftc_ref[NFIX_SC:NFIX]
    any_complete = jnp.max(fl_s[...]) > 0.5

    @pl.when(any_complete)
    def _():
        def group_body(g, _):
            fv = fl_s[g]
            gsum = jnp.sum(fv)

            @pl.when(gsum > 0.5)
            def _():
                lane_ids = lax.broadcasted_iota(jnp.int32, (1, LANE), 1)
                for rr in range(LANE):
                    flag_r = jnp.sum(
                        fv * (lane_ids == rr).astype(jnp.float32))

                    @pl.when(flag_r > 0.5)
                    def _():
                        row = g * LANE + rr
                        pltpu.make_async_copy(
                            x_ref.at[pl.ds(row, 1)],
                            o_ref.at[pl.ds(row, 1)],
                            sem,
                        ).start()
                        pltpu.make_async_copy(
                            x_ref.at[pl.ds(row, 1)],
                            o_ref.at[pl.ds(row, 1)],
                            sem,
                        ).wait()
            return 0

        lax.fori_loop(0, NFIX, group_body, 0)


def _fixup(blend_out, X, mflags_sc, flags_tc):
    return pl.pallas_call(
        _fixup_body,
        in_specs=[
            pl.BlockSpec(memory_space=pl.ANY),
            pl.BlockSpec(memory_space=pl.ANY),
            pl.BlockSpec((S_SC, 64), lambda: (0, 0)),
            pl.BlockSpec((NFIX, LANE), lambda: (0, 0)),
        ],
        out_specs=pl.BlockSpec(memory_space=pl.ANY),
        out_shape=jax.ShapeDtypeStruct((N, D), jnp.float32),
        scratch_shapes=[
            pltpu.VMEM((NFIX, LANE), jnp.float32),
            pltpu.SemaphoreType.DMA,
        ],
        input_output_aliases={0: 0},
    )(blend_out, X, mflags_sc, flags_tc)


def kernel(X, mask, h_views, lowconf_edges, infotrans_edges,
           W1, b1, W2, b2, W3, b3):
    mask_i8 = mask.astype(jnp.int8)
    mflags_sc = _sc_flags()(mask_i8)
    blend_out, flags_tc = _blend(X, mask_i8, b1, W2, b2, W3, b3)
    return _fixup(blend_out, X, mflags_sc, flags_tc)


# speedup vs baseline: 1.0639x; 1.0639x over previous
import functools

import jax
import jax.numpy as jnp
from jax import lax
from jax.experimental import pallas as pl
from jax.experimental.pallas import tpu as pltpu
from jax.experimental.pallas import tpu_sc as plsc

N, D, H = 4096, 1024, 128
EMA = 0.9
LANE = 16
NC, NS = 2, 16
NW = NC * NS

S_SC = 1024
ROWS_PER_W = S_SC // NW
N_TC = N - S_SC

BLKT = 2048
NFIX = N // LANE
NFIX_SC = S_SC // LANE


def _blend_body(x_ref, mask_ref, b1_ref, w2_ref, b2_ref, w3_ref, b3_ref,
                o_ref, flags_ref):
    h1 = jax.nn.relu(b1_ref[...])
    h2 = jax.nn.relu(
        lax.dot_general(h1, w2_ref[...], (((1,), (1,)), ((), ())),
                        precision=lax.Precision.HIGHEST) + b2_ref[...])
    pred = lax.dot_general(h2, w3_ref[...], (((1,), (1,)), ((), ())),
                           precision=lax.Precision.HIGHEST) + b3_ref[...]
    p01 = (1.0 - EMA) * pred
    o_ref[...] = EMA * x_ref[...] + p01
    m = mask_ref[...].astype(jnp.int32)
    complete = jnp.min(m, axis=1, keepdims=True)
    flags_ref[...] = complete.astype(jnp.float32).reshape(BLKT // LANE, LANE)


def _blend(X, mask_i8, b1, W2, b2, W3, b3):
    return pl.pallas_call(
        _blend_body,
        grid=(N // BLKT,),
        in_specs=[
            pl.BlockSpec((BLKT, D), lambda i: (i, 0)),
            pl.BlockSpec((BLKT, D), lambda i: (i, 0)),
            pl.BlockSpec((1, H), lambda i: (0, 0)),
            pl.BlockSpec((H, H), lambda i: (0, 0)),
            pl.BlockSpec((1, H), lambda i: (0, 0)),
            pl.BlockSpec((D, H), lambda i: (0, 0)),
            pl.BlockSpec((1, D), lambda i: (0, 0)),
        ],
        out_specs=[
            pl.BlockSpec((BLKT, D), lambda i: (i, 0)),
            pl.BlockSpec((BLKT // LANE, LANE), lambda i: (i, 0)),
        ],
        out_shape=[
            jax.ShapeDtypeStruct((N, D), jnp.float32),
            jax.ShapeDtypeStruct((N // LANE, LANE), jnp.float32),
        ],
    )(X, mask_i8, b1.reshape(1, H), W2, b2.reshape(1, H), W3,
      b3.reshape(1, D))


def _sc_flags_body(mask_hbm_2d, flags_hbm_2d, m_v, f_v, in_sem):
    wid = lax.axis_index("s") * NC + lax.axis_index("c")
    base = wid * ROWS_PER_W
    mask_hbm = mask_hbm_2d.reshape(N // 8, 8, D)
    flags_hbm = flags_hbm_2d.reshape(S_SC // 8, 8, 64)

    pltpu.async_copy(
        mask_hbm.at[pl.ds(base // 8, ROWS_PER_W // 8)], m_v, in_sem).wait()

    for q in range(ROWS_PER_W // 8):
        for k in range(8):
            vs = [m_v[q, k, pl.ds(w * 64, 64)] for w in range(LANE)]
            while len(vs) > 1:
                vs = [jnp.minimum(vs[i], vs[i + 1])
                      for i in range(0, len(vs), 2)]
            f_v[q, k] = vs[0]
    pltpu.sync_copy(f_v, flags_hbm.at[pl.ds(base // 8, ROWS_PER_W // 8)])


@functools.lru_cache(maxsize=1)
def _sc_flags():
    return pl.kernel(
        _sc_flags_body,
        out_type=jax.ShapeDtypeStruct((S_SC, 64), jnp.int8),
        mesh=plsc.VectorSubcoreMesh(core_axis_name="c", subcore_axis_name="s"),
        scratch_types=[
            pltpu.VMEM((ROWS_PER_W // 8, 8, D), jnp.int8),
            pltpu.VMEM((ROWS_PER_W // 8, 8, 64), jnp.int8),
            pltpu.SemaphoreType.DMA,
        ],
    )


def _fixup_body(blend_ref, x_ref, mf_ref, ftc_ref, o_ref, fl_s, sem):
    mf = mf_ref[...].astype(jnp.int32)
    rowmin = jnp.min(mf, axis=1)
    fl_s[0:NFIX_SC] = rowmin.reshape(NFIX_SC, LANE).astype(jnp.float32)
    fl_s[NFIX_SC:NFIX] = ftc_ref[NFIX_SC:NFIX]
    any_complete = jnp.max(fl_s[...]) > 0.5

    @pl.when(any_complete)
    def _():
        def group_body(g, _):
            fv = fl_s[g]
            gsum = jnp.sum(fv)

            @pl.when(gsum > 0.5)
            def _():
                lane_ids = lax.broadcasted_iota(jnp.int32, (1, LANE), 1)
                for rr in range(LANE):
                    flag_r = jnp.sum(
                        fv * (lane_ids == rr).astype(jnp.float32))

                    @pl.when(flag_r > 0.5)
                    def _():
                        row = g * LANE + rr
                        pltpu.make_async_copy(
                            x_ref.at[pl.ds(row, 1)],
                            o_ref.at[pl.ds(row, 1)],
                            sem,
                        ).start()
                        pltpu.make_async_copy(
                            x_ref.at[pl.ds(row, 1)],
                            o_ref.at[pl.ds(row, 1)],
                            sem,
                        ).wait()
            return 0

        lax.fori_loop(0, NFIX, group_body, 0)


def _fixup(blend_out, X, mflags_sc, flags_tc):
    return pl.pallas_call(
        _fixup_body,
        in_specs=[
            pl.BlockSpec(memory_space=pl.ANY),
            pl.BlockSpec(memory_space=pl.ANY),
            pl.BlockSpec((S_SC, 64), lambda: (0, 0)),
            pl.BlockSpec((NFIX, LANE), lambda: (0, 0)),
        ],
        out_specs=pl.BlockSpec(memory_space=pl.ANY),
        out_shape=jax.ShapeDtypeStruct((N, D), jnp.float32),
        scratch_shapes=[
            pltpu.VMEM((NFIX, LANE), jnp.float32),
            pltpu.SemaphoreType.DMA,
        ],
        input_output_aliases={0: 0},
    )(blend_out, X, mflags_sc, flags_tc)


def kernel(X, mask, h_views, lowconf_edges, infotrans_edges,
           W1, b1, W2, b2, W3, b3):
    mask_i8 = mask.astype(jnp.int8)
    mflags_sc = _sc_flags()(mask_i8)
    blend_out, flags_tc = _blend(X, mask_i8, b1, W2, b2, W3, b3)
    return _fixup(blend_out, X, mflags_sc, flags_tc)
